# Initial kernel scaffold; baseline (speedup 1.0000x reference)
import jax
import jax.numpy as jnp
from jax.experimental import pallas as pl


def kernel(x, edge_index, W1, b1, W2, b2, Wfc, bfc):
    raise NotImplementedError("write your pallas kernel here")


# speedup vs baseline: 12.6787x; 12.6787x over previous
import functools

import jax
import jax.numpy as jnp
from jax import lax
from jax.experimental import pallas as pl
from jax.experimental.pallas import tpu as pltpu
from jax.experimental.pallas import tpu_sc as plsc

F32 = jnp.float32

NC = 2
NS = 16
NW = NC * NS
CH = 128


def _cdiv(a, b):
    return (a + b - 1) // b


def _make_deg_kernel(npad, nchunk):
    rows_per_tile = npad // NS
    nz = rows_per_tile // CH
    mesh = plsc.VectorSubcoreMesh(core_axis_name="c", subcore_axis_name="s")

    @functools.partial(
        pl.kernel,
        mesh=mesh,
        out_type=jax.ShapeDtypeStruct((NC, npad, 16), F32),
        scratch_types=[
            pltpu.VMEM((nchunk, CH), jnp.int32),
            pltpu.VMEM((CH, 16), F32),
            pltpu.VMEM((CH, 16), F32),
            pltpu.VMEM_SHARED((npad, 16), F32),
        ],
    )
    def deg_kernel(dst_hbm, out_hbm, dst_v, ones_v, zero_v, deg_sh):
        cid = lax.axis_index("c")
        sid = lax.axis_index("s")
        wid = cid * NS + sid
        pltpu.sync_copy(dst_hbm.at[wid], dst_v)

        def fill(i, _):
            ones_v[i, :] = jnp.ones((16,), F32)
            zero_v[i, :] = jnp.zeros((16,), F32)
            return _

        lax.fori_loop(0, CH, fill, 0)
        base = sid * rows_per_tile
        for z in range(nz):
            pltpu.sync_copy(zero_v, deg_sh.at[pl.ds(base + z * CH, CH)])
        plsc.subcore_barrier()

        def body(j, _):
            pltpu.sync_copy(ones_v, deg_sh.at[dst_v.at[j]], add=True)
            return _

        lax.fori_loop(0, nchunk, body, 0)
        plsc.subcore_barrier()
        for z in range(nz):
            sl = pl.ds(base + z * CH, CH)
            pltpu.sync_copy(deg_sh.at[sl], out_hbm.at[cid, sl])

    return deg_kernel


def _make_edge_sum_kernel(npad, nchunk):
    rows_per_tile = npad // NS
    nz = rows_per_tile // CH
    mesh = plsc.VectorSubcoreMesh(core_axis_name="c", subcore_axis_name="s")

    @functools.partial(
        pl.kernel,
        mesh=mesh,
        out_type=jax.ShapeDtypeStruct((NC, npad, 128), F32),
        scratch_types=[
            pltpu.VMEM((nchunk, CH), jnp.int32),
            pltpu.VMEM((nchunk, CH), jnp.int32),
            pltpu.VMEM((CH, 128), F32),
            pltpu.VMEM_SHARED((npad, 128), F32),
        ],
    )
    def edge_kernel(y_hbm, src_hbm, dst_hbm, out_hbm, src_v, dst_v, rows_v, z_sh):
        cid = lax.axis_index("c")
        sid = lax.axis_index("s")
        wid = cid * NS + sid
        pltpu.sync_copy(src_hbm.at[wid], src_v)
        pltpu.sync_copy(dst_hbm.at[wid], dst_v)

        def zrow(i, _):
            for k in range(8):
                rows_v[i, pl.ds(k * 16, 16)] = jnp.zeros((16,), F32)
            return _

        lax.fori_loop(0, CH, zrow, 0)
        base = sid * rows_per_tile
        for z in range(nz):
            pltpu.sync_copy(rows_v, z_sh.at[pl.ds(base + z * CH, CH)])
        plsc.subcore_barrier()

        def body(j, _):
            pltpu.sync_copy(y_hbm.at[src_v.at[j]], rows_v)
            pltpu.sync_copy(rows_v, z_sh.at[dst_v.at[j]], add=True)
            return _

        lax.fori_loop(0, nchunk, body, 0)
        plsc.subcore_barrier()
        for z in range(nz):
            sl = pl.ds(base + z * CH, CH)
            pltpu.sync_copy(z_sh.at[sl], out_hbm.at[cid, sl])

    return edge_kernel


BLK = 1280


def _dinv_of(d0, d1):
    deg = d0[:, 0:1] + d1[:, 0:1] + 1.0
    return lax.rsqrt(deg)


def _tc1_body(x_ref, w_ref, d0_ref, d1_ref, y_ref):
    dinv = _dinv_of(d0_ref[...], d1_ref[...])
    xw = jnp.dot(x_ref[...], w_ref[...], preferred_element_type=F32)
    y_ref[...] = xw * dinv


def _tc2_body(z0_ref, z1_ref, y_ref, d0_ref, d1_ref, w_ref, b_ref, o_ref):
    dinv = _dinv_of(d0_ref[...], d1_ref[...])
    h = dinv * (z0_ref[...] + z1_ref[...] + y_ref[...]) + b_ref[...]
    h = jnp.maximum(h, 0.0)
    o_ref[...] = dinv * jnp.dot(h, w_ref[...], preferred_element_type=F32)


def _tc3_body(z0_ref, z1_ref, y_ref, d0_ref, d1_ref, b_ref, wfc_ref, bfc_ref,
              o_ref):
    dinv = _dinv_of(d0_ref[...], d1_ref[...])
    h = dinv * (z0_ref[...] + z1_ref[...] + y_ref[...]) + b_ref[...]
    h = jnp.maximum(h, 0.0)
    o_ref[...] = jnp.sum(h * wfc_ref[...], axis=1, keepdims=True) + bfc_ref[0, 0]


def _row_spec(w):
    return pl.BlockSpec((BLK, w), lambda i: (i, 0))


def _full_spec(shape):
    return pl.BlockSpec(shape, lambda i: (0,) * len(shape))


@jax.jit
def kernel(x, edge_index, W1, b1, W2, b2, Wfc, bfc):
    N, D = x.shape
    E = edge_index.shape[1]

    npad = 2048 * _cdiv(N + 1, 2048)
    nchunk = _cdiv(E, NW * CH)
    ew = nchunk * CH

    src = edge_index[0]
    dst = edge_index[1]
    pad = NW * ew - E
    src_p = jnp.concatenate([src, jnp.zeros((pad,), jnp.int32)]).reshape(NW, nchunk, CH)
    dst_p = jnp.concatenate([dst, jnp.full((pad,), N, jnp.int32)]).reshape(NW, nchunk, CH)

    x_pad = jnp.zeros((npad, D), F32).at[:N].set(x)

    deg_kernel = _make_deg_kernel(npad, nchunk)
    edge_kernel = _make_edge_sum_kernel(npad, nchunk)

    degp = deg_kernel(dst_p)
    d0, d1 = degp[0], degp[1]

    grid = (npad // BLK,)
    y1 = pl.pallas_call(
        _tc1_body,
        grid=grid,
        in_specs=[_row_spec(D), _full_spec((D, D)), _row_spec(16), _row_spec(16)],
        out_specs=_row_spec(D),
        out_shape=jax.ShapeDtypeStruct((npad, D), F32),
    )(x_pad, W1, d0, d1)

    zp1 = edge_kernel(y1, src_p, dst_p)

    y2 = pl.pallas_call(
        _tc2_body,
        grid=grid,
        in_specs=[_row_spec(D), _row_spec(D), _row_spec(D), _row_spec(16),
                  _row_spec(16), _full_spec((D, D)), _full_spec((1, D))],
        out_specs=_row_spec(D),
        out_shape=jax.ShapeDtypeStruct((npad, D), F32),
    )(zp1[0], zp1[1], y1, d0, d1, W2, b1.reshape(1, D))

    zp2 = edge_kernel(y2, src_p, dst_p)

    logits = pl.pallas_call(
        _tc3_body,
        grid=grid,
        in_specs=[_row_spec(D), _row_spec(D), _row_spec(D), _row_spec(16),
                  _row_spec(16), _full_spec((1, D)), _full_spec((1, D)),
                  _full_spec((1, 1))],
        out_specs=_row_spec(1),
        out_shape=jax.ShapeDtypeStruct((npad, 1), F32),
    )(zp2[0], zp2[1], y2, d0, d1, b2.reshape(1, D), Wfc.reshape(1, D),
      bfc.reshape(1, 1))

    return logits[:N, 0]
